# baseline (speedup 1.0000x reference)
import functools
import math

import jax
import jax.numpy as jnp
from jax import lax
from jax.experimental import pallas as pl
from jax.experimental.pallas import tpu as pltpu
from jax.experimental.pallas import tpu_sc as plsc


def _pe_block(x_ref, pos_ref, o_ref, *, scale):
    o_ref[...] = x_ref[...] * scale + pos_ref[...][None, :, :]


def _pe_tc(x, pos_table, tc_rows, block_s):
    batch, seq_len, d_model = x.shape
    scale = math.sqrt(float(d_model))
    grid = (tc_rows // block_s,)
    return pl.pallas_call(
        functools.partial(_pe_block, scale=scale),
        grid=grid,
        in_specs=[
            pl.BlockSpec((batch, block_s, d_model), lambda s: (0, s, 0)),
            pl.BlockSpec((block_s, d_model), lambda s: (s, 0)),
        ],
        out_specs=pl.BlockSpec((batch, block_s, d_model), lambda s: (0, s, 0)),
        out_shape=jax.ShapeDtypeStruct(x.shape, x.dtype),
    )(x, pos_table)


_SC_LANES = 16


def _pe_sc_body(x_hbm, pos_hbm, o_hbm, xbuf, pbuf, *, batch, d_model,
                base_row, rows_per_worker, chunk_rows, scale, num_cores,
                num_subcores):
    wid = lax.axis_index("s") * num_cores + lax.axis_index("c")
    row0 = base_row + wid * rows_per_worker
    out_row0 = wid * rows_per_worker
    n_chunks = rows_per_worker // chunk_rows
    n_lane_blocks = d_model // _SC_LANES

    def chunk_loop(ci, _):
        src_r = row0 + ci * chunk_rows
        dst_r = out_row0 + ci * chunk_rows
        pltpu.sync_copy(pos_hbm.at[pl.ds(src_r, chunk_rows), :], pbuf)

        def batch_loop(b, _):
            pltpu.sync_copy(x_hbm.at[b, pl.ds(src_r, chunk_rows), :], xbuf)

            def row_loop(r, _):
                for c in range(n_lane_blocks):
                    o = c * _SC_LANES
                    xv = xbuf[r, pl.ds(o, _SC_LANES)]
                    pv = pbuf[r, pl.ds(o, _SC_LANES)]
                    xbuf[r, pl.ds(o, _SC_LANES)] = xv * scale + pv
                return 0

            lax.fori_loop(0, chunk_rows, row_loop, 0)
            pltpu.sync_copy(xbuf, o_hbm.at[b, pl.ds(dst_r, chunk_rows), :])
            return 0

        lax.fori_loop(0, batch, batch_loop, 0)
        return 0

    lax.fori_loop(0, n_chunks, chunk_loop, 0)


def _pe_sc(x, pos_table, base_row, sc_rows, chunk_rows):
    batch, seq_len, d_model = x.shape
    scale = math.sqrt(float(d_model))
    info = plsc.get_sparse_core_info()
    num_cores, num_subcores = info.num_cores, info.num_subcores
    n_workers = num_cores * num_subcores
    rows_per_worker = sc_rows // n_workers
    chunk_rows = min(chunk_rows, rows_per_worker)

    mesh = plsc.VectorSubcoreMesh(core_axis_name="c", subcore_axis_name="s")
    body = functools.partial(
        _pe_sc_body,
        batch=batch,
        d_model=d_model,
        base_row=base_row,
        rows_per_worker=rows_per_worker,
        chunk_rows=chunk_rows,
        scale=scale,
        num_cores=num_cores,
        num_subcores=num_subcores,
    )
    return pl.kernel(
        body,
        mesh=mesh,
        out_type=jax.ShapeDtypeStruct((batch, sc_rows, d_model), x.dtype),
        scratch_types=[
            pltpu.VMEM((chunk_rows, d_model), jnp.float32),
            pltpu.VMEM((chunk_rows, d_model), jnp.float32),
        ],
    )(x, pos_table)


@functools.partial(jax.jit, static_argnames=("sc_rows", "block_s", "chunk_rows"))
def _pe(x, pos_table, sc_rows=1024, block_s=1024, chunk_rows=32):
    batch, seq_len, d_model = x.shape
    tc_rows = seq_len - sc_rows
    sc_out = _pe_sc(x, pos_table, tc_rows, sc_rows, chunk_rows)
    tc_out = _pe_tc(x, pos_table, tc_rows, block_s)
    return lax.dynamic_update_slice(tc_out, sc_out, (0, tc_rows, 0))


def kernel(x, pos_table):
    return _pe(x, pos_table)


# speedup vs baseline: 1.3730x; 1.3730x over previous
import functools
import math

import jax
import jax.numpy as jnp
from jax.experimental import pallas as pl


def _pe_block(x_ref, pos_ref, o_ref, *, scale):
    o_ref[...] = x_ref[...] * scale + pos_ref[...][None, :, :]


@functools.partial(jax.jit, static_argnames=("block_s", "block_b"))
def _pe(x, pos_table, block_s=1024, block_b=None):
    batch, seq_len, d_model = x.shape
    scale = math.sqrt(float(d_model))
    block_b = batch if block_b is None else block_b
    grid = (seq_len // block_s, batch // block_b)
    return pl.pallas_call(
        functools.partial(_pe_block, scale=scale),
        grid=grid,
        in_specs=[
            pl.BlockSpec((block_b, block_s, d_model), lambda s, b: (b, s, 0)),
            pl.BlockSpec((block_s, d_model), lambda s, b: (s, 0)),
        ],
        out_specs=pl.BlockSpec((block_b, block_s, d_model), lambda s, b: (b, s, 0)),
        out_shape=jax.ShapeDtypeStruct(x.shape, x.dtype),
    )(x, pos_table)


def kernel(x, pos_table):
    return _pe(x, pos_table)
